# baseline (speedup 1.0000x reference)
import functools

import jax
import jax.numpy as jnp
from jax import lax
from jax.experimental import pallas as pl
from jax.experimental.pallas import tpu as pltpu
from jax.experimental.pallas import tpu_sc as plsc

B = 4096
L = 200
EMBED = 32
HD = EMBED // 2
POS = 200
LANES = 16

NUM_CORES = 2
NUM_SUBCORES = 16
NW = NUM_CORES * NUM_SUBCORES
BPW = B // NW
PB = BPW // 2
C0 = 104
C1 = L - C0

_MESH = plsc.VectorSubcoreMesh(core_axis_name="c", subcore_axis_name="s")
_PARAMS = pltpu.CompilerParams(
    use_tc_tiling_on_sc=False, needs_layout_passes=False)


def _make_pool(with_t3):
  out_type = [
      jax.ShapeDtypeStruct((B, HD), jnp.float32),
      jax.ShapeDtypeStruct((B, HD), jnp.float32),
  ]
  scratch = [
      pltpu.VMEM((L, PB), jnp.int32),
      pltpu.VMEM((L, PB), jnp.int32),
      pltpu.VMEM((L, PB), jnp.float32),
      pltpu.VMEM((PB * L,), jnp.int32),
      pltpu.VMEM((PB * L,), jnp.int32),
      pltpu.VMEM((2, L, HD), jnp.float32),
      pltpu.VMEM((2, L, HD), jnp.float32),
      pltpu.VMEM((PB, HD), jnp.float32),
      pltpu.VMEM((PB, HD), jnp.float32),
      pltpu.SemaphoreType.DMA,
      pltpu.SemaphoreType.DMA,
  ]
  if with_t3:
    out_type.append(jax.ShapeDtypeStruct((B,), jnp.float32))
    scratch += [
        pltpu.VMEM((L, PB), jnp.int32),
        pltpu.VMEM((POS + 1, 1), jnp.float32),
        pltpu.VMEM((PB,), jnp.float32),
    ]

  @functools.partial(
      pl.kernel, mesh=_MESH, compiler_params=_PARAMS,
      out_type=tuple(out_type), scratch_types=scratch)
  def pool(*refs):
    if with_t3:
      (x1h, x2h, x4h, e1h, x3h, e2h, s1h, s2h, s3h,
       x1s, x2s, x4s, x1c, x2c, rows1, rows2, s1a, s2a,
       sem0, sem1, x3s, e2v, s3a) = refs
    else:
      (x1h, x2h, x4h, e1h, s1h, s2h,
       x1s, x2s, x4s, x1c, x2c, rows1, rows2, s1a, s2a,
       sem0, sem1) = refs
    wid = lax.axis_index("s") * NUM_CORES + lax.axis_index("c")
    lane = lax.iota(jnp.int32, LANES)
    zeros_i = jnp.zeros((LANES,), jnp.int32)
    zf = jnp.zeros((LANES,), jnp.float32)
    sems = (sem0, sem1)
    if with_t3:
      pltpu.sync_copy(e2h, e2v)

    for ph in range(2):
      base = wid * BPW + ph * PB
      pltpu.sync_copy(x1h.at[:, pl.ds(base, PB)], x1s)
      pltpu.sync_copy(x2h.at[:, pl.ds(base, PB)], x2s)
      pltpu.sync_copy(x4h.at[:, pl.ds(base, PB)], x4s)
      if with_t3:
        pltpu.sync_copy(x3h.at[:, pl.ds(base, PB)], x3s)

      col_starts = tuple(range(0, L - LANES, LANES)) + (L - LANES,)

      def tr_body(gi, carry):
        gvec = jnp.full((LANES,), gi, jnp.int32)
        for k in col_starts:
          kvec = k + lane
          v1 = plsc.load_gather(x1s, [kvec, gvec])
          v2 = plsc.load_gather(x2s, [kvec, gvec])
          off = pl.multiple_of(gi * L + k, 8)
          x1c[pl.ds(off, LANES)] = v1
          x2c[pl.ds(off, LANES)] = v2
        return carry

      lax.fori_loop(0, PB, tr_body, 0)

      if with_t3:
        for g0 in range(0, PB, LANES):

          def t3_body(j, acc):
            pos = x3s[j, pl.ds(g0, LANES)]
            vals = plsc.load_gather(e2v, [pos, zeros_i])
            return acc + vals

          acc3 = lax.fori_loop(0, L, t3_body, zf, unroll=8)
          s3a[pl.ds(g0, LANES)] = acc3

      def chunk_copies(gi, slot):
        sem = sems[slot]
        out = []
        for (idxc, rows) in ((x1c, rows1), (x2c, rows2)):
          off = pl.multiple_of(gi * L, 8)
          out.append(pltpu.make_async_copy(
              e1h.at[idxc.at[pl.ds(off, C0)]],
              rows.at[slot, pl.ds(0, C0)], sem))
          out.append(pltpu.make_async_copy(
              e1h.at[idxc.at[pl.ds(off + C0, C1)]],
              rows.at[slot, pl.ds(C0, C1)], sem))
        return out

      def fire(gi, slot):
        for cp in chunk_copies(gi, slot):
          cp.start()

      def drain(gi, slot):
        for cp in chunk_copies(gi, slot):
          cp.wait()

      def accumulate(gi, slot):
        gvec = jnp.full((LANES,), gi, jnp.int32)

        def group(jb, jj_lo, accs):
          a1, a2 = accs
          w16 = plsc.load_gather(x4s, [jb + lane, gvec])
          for jj in range(jj_lo, LANES):
            j = jb + jj
            w = jnp.take_along_axis(
                w16, jnp.full((LANES,), jj, jnp.int32), axis=0)
            a1 = a1 + rows1[slot, j, pl.ds(0, LANES)] * w
            a2 = a2 + rows2[slot, j, pl.ds(0, LANES)]
          return (a1, a2)

        def group_body(k, accs):
          return group(pl.multiple_of(k * LANES, LANES), 0, accs)

        accs = lax.fori_loop(0, L // LANES, group_body, (zf, zf))
        a1, a2 = group(L - LANES, LANES - (L % LANES), accs)
        s1a[gi, pl.ds(0, LANES)] = a1
        s2a[gi, pl.ds(0, LANES)] = a2

      fire(0, 0)

      def pair_body(i, carry):
        gi0 = i * 2
        drain(gi0, 0)
        fire(gi0 + 1, 1)
        accumulate(gi0, 0)

        @pl.when(gi0 + 2 < PB)
        def _():
          fire(gi0 + 2, 0)

        drain(gi0 + 1, 1)
        accumulate(gi0 + 1, 1)
        return carry

      lax.fori_loop(0, PB // 2, pair_body, 0)

      pltpu.sync_copy(s1a, s1h.at[pl.ds(base, PB)])
      pltpu.sync_copy(s2a, s2h.at[pl.ds(base, PB)])
      if with_t3:
        pltpu.sync_copy(s3a, s3h.at[pl.ds(base, PB)])

  return pool


_pool_a = _make_pool(with_t3=True)
_pool_b = _make_pool(with_t3=False)


def _tc_tail(s1lo, s1hi, s2lo, s2hi, s3, t1b, w1, bb1, t2b, w2, bb2):
  def body(s1lr, s1hr, s2lr, s2hr, s3r, t1br, w1r, b1r, t2br, w2r, b2r, outr):
    s1 = jnp.concatenate([s1lr[...], s1hr[...]], axis=1)
    s2 = jnp.concatenate([s2lr[...], s2hr[...]], axis=1)
    h1 = jnp.tanh(s1 + t1br[...][None, :])
    h1 = jnp.tanh(
        lax.dot_general(h1, w1r[...], (((1,), (1,)), ((), ())),
                        preferred_element_type=jnp.float32) + b1r[...][None, :])
    h2 = jnp.tanh(s2 + t2br[...][None, :])
    h2 = jnp.tanh(
        lax.dot_general(h2, w2r[...], (((1,), (1,)), ((), ())),
                        preferred_element_type=jnp.float32) + b2r[...][None, :])
    x12 = jax.nn.sigmoid(jnp.sum(h1 * h2, axis=1))
    outr[...] = x12 * jax.nn.sigmoid(s3r[...])

  return pl.pallas_call(
      body,
      out_shape=jax.ShapeDtypeStruct((B,), jnp.float32),
  )(s1lo, s1hi, s2lo, s2hi, s3, t1b, w1, bb1, t2b, w2, bb2)


def kernel(x1, x2, x3, x4, E1, t1_bias1, W1, b1, t2_bias1, W2, b2, E2):
  x1t = x1.astype(jnp.int32).T
  x2t = x2.astype(jnp.int32).T
  x3t = x3.astype(jnp.int32).T
  x4t = x4.T
  s1hi, s2hi = _pool_b(x1t, x2t, x4t, E1[:, HD:])
  s1lo, s2lo, s3 = _pool_a(x1t, x2t, x4t, E1[:, :HD], x3t, E2)
  return _tc_tail(s1lo, s1hi, s2lo, s2hi, s3,
                  t1_bias1, W1, b1, t2_bias1, W2, b2)
